# baseline (speedup 1.0000x reference)
import functools

import jax
import jax.numpy as jnp
from jax import lax
from jax.experimental import pallas as pl
from jax.experimental.pallas import tpu as pltpu
from jax.experimental.pallas import tpu_sc as plsc

B = 1024
PACK = 16
PICKED = 48
SEEN = 45
D = 64
SPD = 512
SROWS = SEEN * PACK
NIDX, WIDX = 6, 120
RSQ_SPD = float(SPD) ** -0.5


NID = SROWS + PICKED + PACK


def _sc_phase(table, idx_all, nb, emit_e0):
    info = plsc.get_sparse_core_info()
    nw = info.num_cores * info.num_subcores
    bpw = nb // nw
    mesh = plsc.VectorSubcoreMesh(core_axis_name="c", subcore_axis_name="s")

    def body(table_h, idx_h, *refs):
        if emit_e0:
            (cce_out, psum_out, praw_out, e0_out, idx_v, srows, prows,
             crows, packbuf, poolbuf, gsem0, gsem1, osem0, osem1) = refs
        else:
            (cce_out, psum_out, praw_out, idx_v, srows, prows,
             crows, packbuf, poolbuf, gsem0, gsem1, osem0, osem1) = refs
        gsems = (gsem0, gsem1)
        osems = (osem0, osem1)
        wid = lax.axis_index("s") * info.num_cores + lax.axis_index("c")
        base = wid * bpw

        if emit_e0:
            @pl.when(wid == 0)
            def _():
                pltpu.sync_copy(table_h.at[0], poolbuf.at[pl.ds(0, D)])
                pltpu.sync_copy(poolbuf.at[pl.ds(0, D)], e0_out.at[0])

        def gather_list(half):
            i0 = half * NID
            res = []
            for k in range(NIDX):
                res.append((idx_v.at[pl.ds(i0 + k * WIDX, WIDX)],
                            srows.at[pl.ds(half * SROWS + k * WIDX, WIDX)]))
            res.append((idx_v.at[pl.ds(i0 + SROWS, PICKED)],
                        prows.at[pl.ds(half * PICKED, PICKED)]))
            res.append((idx_v.at[pl.ds(i0 + SROWS + PICKED, PACK)],
                        crows.at[pl.ds(half * PACK, PACK)]))
            return res

        def stage_and_fire(t, half):
            pltpu.sync_copy(idx_h.at[base + t],
                            idx_v.at[pl.ds(half * NID, NID)])
            for iv, dst in gather_list(half):
                pltpu.async_copy(table_h.at[iv], dst, gsems[half])

        def wait_gathers(half):
            for iv, dst in gather_list(half):
                pltpu.make_async_copy(table_h.at[iv], dst,
                                      gsems[half]).wait()

        def out_list(half, b):
            return [
                (crows.at[pl.ds(half * PACK, PACK)], cce_out.at[b]),
                (poolbuf.at[pl.ds(half * D, D)], praw_out.at[b]),
                (packbuf.at[pl.ds(half * SEEN, SEEN)], psum_out.at[b]),
            ]

        stage_and_fire(0, 0)

        def pair_body(p, _):
            for half in (0, 1):
                t = 2 * p + half
                b = base + t
                oth = 1 - half
                wait_gathers(half)

                @pl.when(t + 1 < bpw)
                def _():
                    stage_and_fire(t + 1, oth)

                @pl.when(t >= 2)
                def _():
                    for src, dst in out_list(half, b - 2):
                        pltpu.make_async_copy(src, dst, osems[half]).wait()

                p0 = half * PICKED
                for c in range(D // 16):
                    acc = prows[p0, pl.ds(c * 16, 16)]
                    for j in range(1, PICKED):
                        acc = acc + prows[p0 + j, pl.ds(c * 16, 16)]
                    poolbuf[pl.ds(half * D + c * 16, 16)] = acc

                r0 = half * SROWS
                s0 = half * SEEN

                def pack_body(s, _):
                    for c in range(D // 16):
                        acc = srows[r0 + s * 16, pl.ds(c * 16, 16)]
                        for j in range(1, PACK):
                            acc = acc + srows[r0 + s * 16 + j,
                                              pl.ds(c * 16, 16)]
                        packbuf[s0 + s, pl.ds(c * 16, 16)] = acc
                    return 0

                lax.fori_loop(0, SEEN, pack_body, 0)

                for src, dst in out_list(half, b):
                    pltpu.async_copy(src, dst, osems[half])
            return 0

        lax.fori_loop(0, bpw // 2, pair_body, 0)
        for half in (0, 1):
            b_last = base + bpw - 2 + half
            for src, dst in out_list(half, b_last):
                pltpu.make_async_copy(src, dst, osems[half]).wait()

    outs = [jax.ShapeDtypeStruct((nb, PACK, D), jnp.float32),
            jax.ShapeDtypeStruct((nb, SEEN, D), jnp.float32),
            jax.ShapeDtypeStruct((nb, D), jnp.float32)]
    if emit_e0:
        outs.append(jax.ShapeDtypeStruct((1, D), jnp.float32))
    kern = pl.kernel(
        body,
        out_type=outs,
        mesh=mesh,
        scratch_types=[
            pltpu.VMEM((2 * NID,), jnp.int32),
            pltpu.VMEM((2 * SROWS, D), jnp.float32),
            pltpu.VMEM((2 * PICKED, D), jnp.float32),
            pltpu.VMEM((2 * PACK, D), jnp.float32),
            pltpu.VMEM((2 * SEEN, D), jnp.float32),
            pltpu.VMEM((2 * D,), jnp.float32),
            pltpu.SemaphoreType.DMA,
            pltpu.SemaphoreType.DMA,
            pltpu.SemaphoreType.DMA,
            pltpu.SemaphoreType.DMA,
        ],
        compiler_params=pltpu.CompilerParams(use_tc_tiling_on_sc=False),
    )
    return kern(table, idx_all)


def _sc_phase_split(table, idx_all):
    h = B // 2
    cce1, psums1, praw1, e0row = _sc_phase(table, idx_all[:h], h, True)
    cce2, psums2, praw2 = _sc_phase(table, idx_all[h:], h, False)
    return (cce1, psums1, praw1), (cce2, psums2, praw2), e0row


def _tca_body(sp_ref, pool_ref, sidx45_ref, scw_ref, cidx45_ref, cw_ref,
              postab_ref, swt_ref, m_out, misc_out, pos_out):
    bb = sp_ref.shape[0]
    spb = (sp_ref[...] > 0).astype(jnp.float32)
    row_seg = lax.broadcasted_iota(jnp.int32, (SROWS, SEEN), 0) // PACK
    seg = lax.broadcasted_iota(jnp.int32, (SROWS, SEEN), 1)
    sel = (row_seg == seg).astype(jnp.float32)
    cnt = jnp.dot(spb, sel, preferred_element_type=jnp.float32)
    smask = (cnt > 0).astype(jnp.float32)
    m = smask / (cnt + 1e-9)
    kcorr = jnp.sum(m * (float(PACK) - cnt), axis=1, keepdims=True)
    smask_sum = jnp.sum(smask, axis=1, keepdims=True)

    pcnt = jnp.sum((pool_ref[...] > 0).astype(jnp.float32), axis=1,
                   keepdims=True)
    pmul = (pcnt > 0).astype(jnp.float32) / (pcnt + 1e-9)

    sidx = sidx45_ref[...].reshape(bb, SEEN, 4)
    wf = scw_ref[...].reshape(bb, SEEN, 4) * smask[:, :, None]
    iota45 = lax.broadcasted_iota(jnp.int32, (1, 1, SEEN), 2)
    c45 = jnp.zeros((bb, SEEN), jnp.float32)
    for j in range(4):
        eq = (sidx[:, :, j:j + 1] == iota45).astype(jnp.float32)
        c45 = c45 + jnp.sum(eq * wf[:, :, j:j + 1], axis=1)
    pos_out[...] = jnp.dot(c45, postab_ref[...],
                           preferred_element_type=jnp.float32)

    cidx = cidx45_ref[...]
    cwv = cw_ref[...]
    iota45b = lax.broadcasted_iota(jnp.int32, (1, 45), 1)
    csw = jnp.zeros((bb, 45), jnp.float32)
    for j in range(4):
        eq = (cidx[:, j:j + 1] == iota45b).astype(jnp.float32)
        csw = csw + eq * cwv[:, j:j + 1]
    sw_lin = jnp.dot(csw, swt_ref[...],
                     preferred_element_type=jnp.float32)
    sw = jnp.maximum(sw_lin, 0.0) + jnp.log1p(jnp.exp(-jnp.abs(sw_lin)))

    m_out[...] = m
    misc_out[...] = jnp.concatenate(
        [kcorr, smask_sum, pcnt, pmul, sw], axis=1)


def _tca_phase(spf, pool_i, sidx45, scw, cidx45, cw, postab, swt):
    bb = 128
    grid = (B // bb,)
    bspecs = [
        pl.BlockSpec((bb, SROWS), lambda i: (i, 0)),
        pl.BlockSpec((bb, PICKED), lambda i: (i, 0)),
        pl.BlockSpec((bb, SEEN * 4), lambda i: (i, 0)),
        pl.BlockSpec((bb, SEEN * 4), lambda i: (i, 0)),
        pl.BlockSpec((bb, 4), lambda i: (i, 0)),
        pl.BlockSpec((bb, 4), lambda i: (i, 0)),
        pl.BlockSpec((SEEN, SPD), lambda i: (0, 0)),
        pl.BlockSpec((SEEN, 3), lambda i: (0, 0)),
    ]
    return pl.pallas_call(
        _tca_body,
        grid=grid,
        in_specs=bspecs,
        out_specs=[pl.BlockSpec((bb, SEEN), lambda i: (i, 0)),
                   pl.BlockSpec((bb, 7), lambda i: (i, 0)),
                   pl.BlockSpec((bb, SPD), lambda i: (i, 0))],
        out_shape=[jax.ShapeDtypeStruct((B, SEEN), jnp.float32),
                   jax.ShapeDtypeStruct((B, 7), jnp.float32),
                   jax.ShapeDtypeStruct((B, SPD), jnp.float32)],
    )(spf, pool_i, sidx45, scw, cidx45, cw, postab, swt)


def _tcb_body(cce_ref, psum_ref, praw_ref, e0_ref, m_ref, misc_ref, pos_ref,
              cc_ref, wpack_ref, bpack_ref, wcst_ref, wrate_ref, brate_ref,
              out_ref):
    e0 = e0_ref[...]
    misc = misc_ref[...]
    kcorr, smask_sum = misc[:, 0:1], misc[:, 1:2]
    pcnt, pmul = misc[:, 2:3], misc[:, 3:4]
    sw0, sw1, sw2 = misc[:, 4:5], misc[:, 5:6], misc[:, 6:7]

    pool_ctx = (praw_ref[...] - (float(PICKED) - pcnt) * e0) * pmul

    m = m_ref[...]
    seen_wsum = jnp.sum(psum_ref[...] * m[:, :, None], axis=1) - kcorr * e0

    a = jnp.dot(seen_wsum, wpack_ref[...],
                preferred_element_type=jnp.float32)
    seen_ctx = (a + bpack_ref[...] * smask_sum + pos_ref[...]) / (
        smask_sum + 1e-9)
    v = jnp.dot(seen_ctx, wcst_ref[...],
                preferred_element_type=jnp.float32)

    u = (sw0 * 0.125) * pool_ctx + (sw1 * RSQ_SPD) * v \
        + sw2 * wrate_ref[...]

    cce = cce_ref[...]
    scores = jnp.sum(cce * u[:, None, :], axis=2) + sw2 * brate_ref[0, 0]
    mask = (cc_ref[...] > 0).astype(jnp.float32)
    out_ref[...] = scores * mask


def _tcb_phase(cce, psums, praw, e0row, m, misc, pos, cc,
               wpack, bpack, wcst, wrate, brate):
    nb = cce.shape[0]
    bb = 128
    grid = (nb // bb,)
    bspecs = [
        pl.BlockSpec((bb, PACK, D), lambda i: (i, 0, 0)),
        pl.BlockSpec((bb, SEEN, D), lambda i: (i, 0, 0)),
        pl.BlockSpec((bb, D), lambda i: (i, 0)),
        pl.BlockSpec((1, D), lambda i: (0, 0)),
        pl.BlockSpec((bb, SEEN), lambda i: (i, 0)),
        pl.BlockSpec((bb, 7), lambda i: (i, 0)),
        pl.BlockSpec((bb, SPD), lambda i: (i, 0)),
        pl.BlockSpec((bb, PACK), lambda i: (i, 0)),
        pl.BlockSpec((D, SPD), lambda i: (0, 0)),
        pl.BlockSpec((1, SPD), lambda i: (0, 0)),
        pl.BlockSpec((SPD, D), lambda i: (0, 0)),
        pl.BlockSpec((1, D), lambda i: (0, 0)),
        pl.BlockSpec((1, 1), lambda i: (0, 0)),
    ]
    return pl.pallas_call(
        _tcb_body,
        grid=grid,
        in_specs=bspecs,
        out_specs=pl.BlockSpec((bb, PACK), lambda i: (i, 0)),
        out_shape=jax.ShapeDtypeStruct((nb, PACK), jnp.float32),
    )(cce, psums, praw, e0row, m, misc, pos, cc,
      wpack, bpack, wcst, wrate, brate)


def kernel(card_choices, pool, seen_packs, seen_coords, seen_coord_weights,
           coords, coord_weights, card_embeddings, W_pack, b_pack,
           pack_pos_table, W_card_seen, W_rate, b_rate,
           sublayer_weight_table):
    cc = card_choices.astype(jnp.int32)
    pool_i = pool.astype(jnp.int32)
    spf = seen_packs.astype(jnp.int32).reshape(B, SROWS)
    idx_all = jnp.concatenate([spf, pool_i, cc], axis=1)

    (sc1, sc2, e0row) = _sc_phase_split(card_embeddings, idx_all)

    sidx45 = (seen_coords[..., 0] * 15 + seen_coords[..., 1]) \
        .astype(jnp.int32).reshape(B, SEEN * 4)
    cidx45 = (coords[..., 0] * 15 + coords[..., 1]).astype(jnp.int32)
    scw = seen_coord_weights.reshape(B, SEEN * 4)

    m, misc, pos = _tca_phase(spf, pool_i, sidx45, scw, cidx45, coord_weights,
                              pack_pos_table, sublayer_weight_table)

    h = B // 2
    wargs = (W_pack, b_pack.reshape(1, SPD), W_card_seen.T,
             W_rate[:, 0].reshape(1, D), b_rate.reshape(1, 1))
    s1 = _tcb_phase(sc1[0], sc1[1], sc1[2], e0row, m[:h], misc[:h], pos[:h],
                    cc[:h], *wargs)
    s2 = _tcb_phase(sc2[0], sc2[1], sc2[2], e0row, m[h:], misc[h:], pos[h:],
                    cc[h:], *wargs)
    return jnp.concatenate([s1, s2], axis=0)


# speedup vs baseline: 1.0739x; 1.0739x over previous
import functools

import jax
import jax.numpy as jnp
from jax import lax
from jax.experimental import pallas as pl
from jax.experimental.pallas import tpu as pltpu
from jax.experimental.pallas import tpu_sc as plsc

B = 1024
PACK = 16
PICKED = 48
SEEN = 45
D = 64
SPD = 512
SROWS = SEEN * PACK
NIDX, WIDX = 6, 120
RSQ_SPD = float(SPD) ** -0.5


NID = SROWS + PICKED + PACK


def _sc_phase(table, idx_all, nb, emit_e0):
    info = plsc.get_sparse_core_info()
    nw = info.num_cores * info.num_subcores
    bpw = nb // nw
    mesh = plsc.VectorSubcoreMesh(core_axis_name="c", subcore_axis_name="s")

    def body(table_h, idx_h, *refs):
        if emit_e0:
            (cce_out, psum_out, praw_out, e0_out, idx_v, srows, pcrows,
             packbuf, poolbuf, gsem0, gsem1, osem0, osem1) = refs
        else:
            (cce_out, psum_out, praw_out, idx_v, srows, pcrows,
             packbuf, poolbuf, gsem0, gsem1, osem0, osem1) = refs
        gsems = (gsem0, gsem1)
        osems = (osem0, osem1)
        wid = lax.axis_index("s") * info.num_cores + lax.axis_index("c")
        base = wid * bpw

        if emit_e0:
            @pl.when(wid == 0)
            def _():
                pltpu.sync_copy(table_h.at[0], poolbuf.at[pl.ds(0, D)])
                pltpu.sync_copy(poolbuf.at[pl.ds(0, D)], e0_out.at[0])

        pltpu.sync_copy(idx_h.at[pl.ds(base * NID, bpw * NID)], idx_v)

        def gather_list(t, half):
            i0 = t * NID
            res = []
            for k in range(NIDX):
                res.append((idx_v.at[pl.ds(i0 + k * WIDX, WIDX)],
                            srows.at[pl.ds(half * SROWS + k * WIDX, WIDX)]))
            res.append((idx_v.at[pl.ds(i0 + SROWS, PICKED + PACK)],
                        pcrows.at[pl.ds(half * (PICKED + PACK),
                                        PICKED + PACK)]))
            return res

        def fire_gathers(t, half):
            for iv, dst in gather_list(t, half):
                pltpu.async_copy(table_h.at[iv], dst, gsems[half])

        def wait_gathers(t, half):
            for iv, dst in gather_list(t, half):
                pltpu.make_async_copy(table_h.at[iv], dst,
                                      gsems[half]).wait()

        def out_list(half, b):
            c0 = half * (PICKED + PACK) + PICKED
            return [
                (pcrows.at[pl.ds(c0, PACK)], cce_out.at[b]),
                (poolbuf.at[pl.ds(half * D, D)], praw_out.at[b]),
                (packbuf.at[pl.ds(half * SEEN, SEEN)], psum_out.at[b]),
            ]

        fire_gathers(0, 0)

        def pair_body(p, _):
            for half in (0, 1):
                t = 2 * p + half
                b = base + t
                oth = 1 - half
                wait_gathers(t, half)

                @pl.when(t + 1 < bpw)
                def _():
                    fire_gathers(t + 1, oth)

                @pl.when(t >= 2)
                def _():
                    for src, dst in out_list(half, b - 2):
                        pltpu.make_async_copy(src, dst, osems[half]).wait()

                p0 = half * (PICKED + PACK)
                for c in range(D // 16):
                    acc = pcrows[p0, pl.ds(c * 16, 16)]
                    for j in range(1, PICKED):
                        acc = acc + pcrows[p0 + j, pl.ds(c * 16, 16)]
                    poolbuf[pl.ds(half * D + c * 16, 16)] = acc

                r0 = half * SROWS
                s0 = half * SEEN

                def pack_body(s, _):
                    for c in range(D // 16):
                        acc = srows[r0 + s * 16, pl.ds(c * 16, 16)]
                        for j in range(1, PACK):
                            acc = acc + srows[r0 + s * 16 + j,
                                              pl.ds(c * 16, 16)]
                        packbuf[s0 + s, pl.ds(c * 16, 16)] = acc
                    return 0

                lax.fori_loop(0, SEEN, pack_body, 0)

                for src, dst in out_list(half, b):
                    pltpu.async_copy(src, dst, osems[half])
            return 0

        lax.fori_loop(0, bpw // 2, pair_body, 0)
        for half in (0, 1):
            b_last = base + bpw - 2 + half
            for src, dst in out_list(half, b_last):
                pltpu.make_async_copy(src, dst, osems[half]).wait()

    outs = [jax.ShapeDtypeStruct((nb, PACK, D), jnp.float32),
            jax.ShapeDtypeStruct((nb, SEEN, D), jnp.float32),
            jax.ShapeDtypeStruct((nb, D), jnp.float32)]
    if emit_e0:
        outs.append(jax.ShapeDtypeStruct((1, D), jnp.float32))
    kern = pl.kernel(
        body,
        out_type=outs,
        mesh=mesh,
        scratch_types=[
            pltpu.VMEM((nb // nw * NID,), jnp.int32),
            pltpu.VMEM((2 * SROWS, D), jnp.float32),
            pltpu.VMEM((2 * (PICKED + PACK), D), jnp.float32),
            pltpu.VMEM((2 * SEEN, D), jnp.float32),
            pltpu.VMEM((2 * D,), jnp.float32),
            pltpu.SemaphoreType.DMA,
            pltpu.SemaphoreType.DMA,
            pltpu.SemaphoreType.DMA,
            pltpu.SemaphoreType.DMA,
        ],
        compiler_params=pltpu.CompilerParams(use_tc_tiling_on_sc=False),
    )
    return kern(table, idx_all.reshape(-1))


def _sc_phase_split(table, idx_all):
    h = B // 2
    cce1, psums1, praw1, e0row = _sc_phase(table, idx_all[:h], h, True)
    cce2, psums2, praw2 = _sc_phase(table, idx_all[h:], h, False)
    return (cce1, psums1, praw1), (cce2, psums2, praw2), e0row


def _tca_body(sp_ref, pool_ref, sidx45_ref, scw_ref, cidx45_ref, cw_ref,
              postab_ref, swt_ref, m_out, misc_out, pos_out):
    bb = sp_ref.shape[0]
    spb = (sp_ref[...] > 0).astype(jnp.float32)
    row_seg = lax.broadcasted_iota(jnp.int32, (SROWS, SEEN), 0) // PACK
    seg = lax.broadcasted_iota(jnp.int32, (SROWS, SEEN), 1)
    sel = (row_seg == seg).astype(jnp.float32)
    cnt = jnp.dot(spb, sel, preferred_element_type=jnp.float32)
    smask = (cnt > 0).astype(jnp.float32)
    m = smask / (cnt + 1e-9)
    kcorr = jnp.sum(m * (float(PACK) - cnt), axis=1, keepdims=True)
    smask_sum = jnp.sum(smask, axis=1, keepdims=True)

    pcnt = jnp.sum((pool_ref[...] > 0).astype(jnp.float32), axis=1,
                   keepdims=True)
    pmul = (pcnt > 0).astype(jnp.float32) / (pcnt + 1e-9)

    sidx = sidx45_ref[...].reshape(bb, SEEN, 4)
    wf = scw_ref[...].reshape(bb, SEEN, 4) * smask[:, :, None]
    iota45 = lax.broadcasted_iota(jnp.int32, (1, 1, SEEN), 2)
    c45 = jnp.zeros((bb, SEEN), jnp.float32)
    for j in range(4):
        eq = (sidx[:, :, j:j + 1] == iota45).astype(jnp.float32)
        c45 = c45 + jnp.sum(eq * wf[:, :, j:j + 1], axis=1)
    pos_out[...] = jnp.dot(c45, postab_ref[...],
                           preferred_element_type=jnp.float32)

    cidx = cidx45_ref[...]
    cwv = cw_ref[...]
    iota45b = lax.broadcasted_iota(jnp.int32, (1, 45), 1)
    csw = jnp.zeros((bb, 45), jnp.float32)
    for j in range(4):
        eq = (cidx[:, j:j + 1] == iota45b).astype(jnp.float32)
        csw = csw + eq * cwv[:, j:j + 1]
    sw_lin = jnp.dot(csw, swt_ref[...],
                     preferred_element_type=jnp.float32)
    sw = jnp.maximum(sw_lin, 0.0) + jnp.log1p(jnp.exp(-jnp.abs(sw_lin)))

    m_out[...] = m
    misc_out[...] = jnp.concatenate(
        [kcorr, smask_sum, pcnt, pmul, sw], axis=1)


def _tca_phase(spf, pool_i, sidx45, scw, cidx45, cw, postab, swt):
    bb = 128
    grid = (B // bb,)
    bspecs = [
        pl.BlockSpec((bb, SROWS), lambda i: (i, 0)),
        pl.BlockSpec((bb, PICKED), lambda i: (i, 0)),
        pl.BlockSpec((bb, SEEN * 4), lambda i: (i, 0)),
        pl.BlockSpec((bb, SEEN * 4), lambda i: (i, 0)),
        pl.BlockSpec((bb, 4), lambda i: (i, 0)),
        pl.BlockSpec((bb, 4), lambda i: (i, 0)),
        pl.BlockSpec((SEEN, SPD), lambda i: (0, 0)),
        pl.BlockSpec((SEEN, 3), lambda i: (0, 0)),
    ]
    return pl.pallas_call(
        _tca_body,
        grid=grid,
        in_specs=bspecs,
        out_specs=[pl.BlockSpec((bb, SEEN), lambda i: (i, 0)),
                   pl.BlockSpec((bb, 7), lambda i: (i, 0)),
                   pl.BlockSpec((bb, SPD), lambda i: (i, 0))],
        out_shape=[jax.ShapeDtypeStruct((B, SEEN), jnp.float32),
                   jax.ShapeDtypeStruct((B, 7), jnp.float32),
                   jax.ShapeDtypeStruct((B, SPD), jnp.float32)],
    )(spf, pool_i, sidx45, scw, cidx45, cw, postab, swt)


def _tcb_body(cce_ref, psum_ref, praw_ref, e0_ref, m_ref, misc_ref, pos_ref,
              cc_ref, wpack_ref, bpack_ref, wcst_ref, wrate_ref, brate_ref,
              out_ref):
    e0 = e0_ref[...]
    misc = misc_ref[...]
    kcorr, smask_sum = misc[:, 0:1], misc[:, 1:2]
    pcnt, pmul = misc[:, 2:3], misc[:, 3:4]
    sw0, sw1, sw2 = misc[:, 4:5], misc[:, 5:6], misc[:, 6:7]

    pool_ctx = (praw_ref[...] - (float(PICKED) - pcnt) * e0) * pmul

    m = m_ref[...]
    seen_wsum = jnp.sum(psum_ref[...] * m[:, :, None], axis=1) - kcorr * e0

    a = jnp.dot(seen_wsum, wpack_ref[...],
                preferred_element_type=jnp.float32)
    seen_ctx = (a + bpack_ref[...] * smask_sum + pos_ref[...]) / (
        smask_sum + 1e-9)
    v = jnp.dot(seen_ctx, wcst_ref[...],
                preferred_element_type=jnp.float32)

    u = (sw0 * 0.125) * pool_ctx + (sw1 * RSQ_SPD) * v \
        + sw2 * wrate_ref[...]

    cce = cce_ref[...]
    scores = jnp.sum(cce * u[:, None, :], axis=2) + sw2 * brate_ref[0, 0]
    mask = (cc_ref[...] > 0).astype(jnp.float32)
    out_ref[...] = scores * mask


def _tcb_phase(cce, psums, praw, e0row, m, misc, pos, cc,
               wpack, bpack, wcst, wrate, brate):
    nb = cce.shape[0]
    bb = 128
    grid = (nb // bb,)
    bspecs = [
        pl.BlockSpec((bb, PACK, D), lambda i: (i, 0, 0)),
        pl.BlockSpec((bb, SEEN, D), lambda i: (i, 0, 0)),
        pl.BlockSpec((bb, D), lambda i: (i, 0)),
        pl.BlockSpec((1, D), lambda i: (0, 0)),
        pl.BlockSpec((bb, SEEN), lambda i: (i, 0)),
        pl.BlockSpec((bb, 7), lambda i: (i, 0)),
        pl.BlockSpec((bb, SPD), lambda i: (i, 0)),
        pl.BlockSpec((bb, PACK), lambda i: (i, 0)),
        pl.BlockSpec((D, SPD), lambda i: (0, 0)),
        pl.BlockSpec((1, SPD), lambda i: (0, 0)),
        pl.BlockSpec((SPD, D), lambda i: (0, 0)),
        pl.BlockSpec((1, D), lambda i: (0, 0)),
        pl.BlockSpec((1, 1), lambda i: (0, 0)),
    ]
    return pl.pallas_call(
        _tcb_body,
        grid=grid,
        in_specs=bspecs,
        out_specs=pl.BlockSpec((bb, PACK), lambda i: (i, 0)),
        out_shape=jax.ShapeDtypeStruct((nb, PACK), jnp.float32),
    )(cce, psums, praw, e0row, m, misc, pos, cc,
      wpack, bpack, wcst, wrate, brate)


def kernel(card_choices, pool, seen_packs, seen_coords, seen_coord_weights,
           coords, coord_weights, card_embeddings, W_pack, b_pack,
           pack_pos_table, W_card_seen, W_rate, b_rate,
           sublayer_weight_table):
    cc = card_choices.astype(jnp.int32)
    pool_i = pool.astype(jnp.int32)
    spf = seen_packs.astype(jnp.int32).reshape(B, SROWS)
    idx_all = jnp.concatenate([spf, pool_i, cc], axis=1)

    (sc1, sc2, e0row) = _sc_phase_split(card_embeddings, idx_all)

    sidx45 = (seen_coords[..., 0] * 15 + seen_coords[..., 1]) \
        .astype(jnp.int32).reshape(B, SEEN * 4)
    cidx45 = (coords[..., 0] * 15 + coords[..., 1]).astype(jnp.int32)
    scw = seen_coord_weights.reshape(B, SEEN * 4)

    m, misc, pos = _tca_phase(spf, pool_i, sidx45, scw, cidx45, coord_weights,
                              pack_pos_table, sublayer_weight_table)

    h = B // 2
    wargs = (W_pack, b_pack.reshape(1, SPD), W_card_seen.T,
             W_rate[:, 0].reshape(1, D), b_rate.reshape(1, 1))
    s1 = _tcb_phase(sc1[0], sc1[1], sc1[2], e0row, m[:h], misc[:h], pos[:h],
                    cc[:h], *wargs)
    s2 = _tcb_phase(sc2[0], sc2[1], sc2[2], e0row, m[h:], misc[h:], pos[h:],
                    cc[h:], *wargs)
    return jnp.concatenate([s1, s2], axis=0)


# speedup vs baseline: 1.0752x; 1.0013x over previous
import functools

import jax
import jax.numpy as jnp
from jax import lax
from jax.experimental import pallas as pl
from jax.experimental.pallas import tpu as pltpu
from jax.experimental.pallas import tpu_sc as plsc

B = 1024
PACK = 16
PICKED = 48
SEEN = 45
D = 64
SPD = 512
SROWS = SEEN * PACK
NIDX, WIDX = 6, 120
RSQ_SPD = float(SPD) ** -0.5


NID = SROWS + PICKED + PACK


def _sc_phase(table, idx_all, nb, emit_e0):
    info = plsc.get_sparse_core_info()
    nw = info.num_cores * info.num_subcores
    bpw = nb // nw
    mesh = plsc.VectorSubcoreMesh(core_axis_name="c", subcore_axis_name="s")

    def body(table_h, idx_h, *refs):
        if emit_e0:
            (cce_out, psum_out, praw_out, e0_out, idx_v, rows_v,
             packbuf, poolbuf, gsem0, gsem1, osem0, osem1) = refs
        else:
            (cce_out, psum_out, praw_out, idx_v, rows_v,
             packbuf, poolbuf, gsem0, gsem1, osem0, osem1) = refs
        gsems = (gsem0, gsem1)
        osems = (osem0, osem1)
        wid = lax.axis_index("s") * info.num_cores + lax.axis_index("c")
        base = wid * bpw

        if emit_e0:
            @pl.when(wid == 0)
            def _():
                pltpu.sync_copy(table_h.at[0], poolbuf.at[pl.ds(0, D)])
                pltpu.sync_copy(poolbuf.at[pl.ds(0, D)], e0_out.at[0])

        pltpu.sync_copy(idx_h.at[pl.ds(base * NID, bpw * NID)], idx_v)

        def gather_list(t, half):
            i0 = t * NID
            r0 = half * NID
            res = []
            for k in range(NIDX):
                res.append((idx_v.at[pl.ds(i0 + k * WIDX, WIDX)],
                            rows_v.at[pl.ds(r0 + k * WIDX, WIDX)]))
            res.append((idx_v.at[pl.ds(i0 + SROWS, PICKED + PACK)],
                        rows_v.at[pl.ds(r0 + SROWS, PICKED + PACK)]))
            return res

        def fire_gathers(t, half):
            for iv, dst in gather_list(t, half):
                pltpu.async_copy(table_h.at[iv], dst, gsems[half])

        def wait_gathers(t, half):
            for iv, dst in gather_list(t, half):
                pltpu.make_async_copy(table_h.at[iv], dst,
                                      gsems[half]).wait()

        def out_list(half, b):
            return [
                (rows_v.at[pl.ds(half * NID + SROWS + PICKED, PACK)],
                 cce_out.at[b]),
                (poolbuf.at[pl.ds(half * D, D)], praw_out.at[b]),
                (packbuf.at[pl.ds(half * SEEN, SEEN)], psum_out.at[b]),
            ]

        fire_gathers(0, 0)

        def pair_body(p, _):
            for half in (0, 1):
                t = 2 * p + half
                b = base + t
                oth = 1 - half
                wait_gathers(t, half)

                @pl.when(t + 1 < bpw)
                def _():
                    fire_gathers(t + 1, oth)

                @pl.when(t >= 2)
                def _():
                    for src, dst in out_list(half, b - 2):
                        pltpu.make_async_copy(src, dst, osems[half]).wait()

                p0 = half * NID + SROWS
                for c in range(D // 16):
                    acc = rows_v[p0, pl.ds(c * 16, 16)]
                    for j in range(1, PICKED):
                        acc = acc + rows_v[p0 + j, pl.ds(c * 16, 16)]
                    poolbuf[pl.ds(half * D + c * 16, 16)] = acc

                r0 = half * NID
                s0 = half * SEEN

                def pack_body(s, _):
                    for c in range(D // 16):
                        acc = rows_v[r0 + s * 16, pl.ds(c * 16, 16)]
                        for j in range(1, PACK):
                            acc = acc + rows_v[r0 + s * 16 + j,
                                               pl.ds(c * 16, 16)]
                        packbuf[s0 + s, pl.ds(c * 16, 16)] = acc
                    return 0

                lax.fori_loop(0, SEEN, pack_body, 0)

                for src, dst in out_list(half, b):
                    pltpu.async_copy(src, dst, osems[half])
            return 0

        lax.fori_loop(0, bpw // 2, pair_body, 0)
        for half in (0, 1):
            b_last = base + bpw - 2 + half
            for src, dst in out_list(half, b_last):
                pltpu.make_async_copy(src, dst, osems[half]).wait()

    outs = [jax.ShapeDtypeStruct((nb, PACK, D), jnp.float32),
            jax.ShapeDtypeStruct((nb, SEEN, D), jnp.float32),
            jax.ShapeDtypeStruct((nb, D), jnp.float32)]
    if emit_e0:
        outs.append(jax.ShapeDtypeStruct((1, D), jnp.float32))
    kern = pl.kernel(
        body,
        out_type=outs,
        mesh=mesh,
        scratch_types=[
            pltpu.VMEM((nb // nw * NID,), jnp.int32),
            pltpu.VMEM((2 * NID, D), jnp.float32),
            pltpu.VMEM((2 * SEEN, D), jnp.float32),
            pltpu.VMEM((2 * D,), jnp.float32),
            pltpu.SemaphoreType.DMA,
            pltpu.SemaphoreType.DMA,
            pltpu.SemaphoreType.DMA,
            pltpu.SemaphoreType.DMA,
        ],
        compiler_params=pltpu.CompilerParams(use_tc_tiling_on_sc=False),
    )
    return kern(table, idx_all.reshape(-1))


def _sc_phase_split(table, idx_all):
    h = B // 2
    cce1, psums1, praw1, e0row = _sc_phase(table, idx_all[:h], h, True)
    cce2, psums2, praw2 = _sc_phase(table, idx_all[h:], h, False)
    return (cce1, psums1, praw1), (cce2, psums2, praw2), e0row


def _tca_body(sp_ref, pool_ref, sidx45_ref, scw_ref, cidx45_ref, cw_ref,
              postab_ref, swt_ref, m_out, misc_out, pos_out):
    bb = sp_ref.shape[0]
    spb = (sp_ref[...] > 0).astype(jnp.float32)
    row_seg = lax.broadcasted_iota(jnp.int32, (SROWS, SEEN), 0) // PACK
    seg = lax.broadcasted_iota(jnp.int32, (SROWS, SEEN), 1)
    sel = (row_seg == seg).astype(jnp.float32)
    cnt = jnp.dot(spb, sel, preferred_element_type=jnp.float32)
    smask = (cnt > 0).astype(jnp.float32)
    m = smask / (cnt + 1e-9)
    kcorr = jnp.sum(m * (float(PACK) - cnt), axis=1, keepdims=True)
    smask_sum = jnp.sum(smask, axis=1, keepdims=True)

    pcnt = jnp.sum((pool_ref[...] > 0).astype(jnp.float32), axis=1,
                   keepdims=True)
    pmul = (pcnt > 0).astype(jnp.float32) / (pcnt + 1e-9)

    sidx = sidx45_ref[...].reshape(bb, SEEN, 4)
    wf = scw_ref[...].reshape(bb, SEEN, 4) * smask[:, :, None]
    iota45 = lax.broadcasted_iota(jnp.int32, (1, 1, SEEN), 2)
    c45 = jnp.zeros((bb, SEEN), jnp.float32)
    for j in range(4):
        eq = (sidx[:, :, j:j + 1] == iota45).astype(jnp.float32)
        c45 = c45 + jnp.sum(eq * wf[:, :, j:j + 1], axis=1)
    pos_out[...] = jnp.dot(c45, postab_ref[...],
                           preferred_element_type=jnp.float32)

    cidx = cidx45_ref[...]
    cwv = cw_ref[...]
    iota45b = lax.broadcasted_iota(jnp.int32, (1, 45), 1)
    csw = jnp.zeros((bb, 45), jnp.float32)
    for j in range(4):
        eq = (cidx[:, j:j + 1] == iota45b).astype(jnp.float32)
        csw = csw + eq * cwv[:, j:j + 1]
    sw_lin = jnp.dot(csw, swt_ref[...],
                     preferred_element_type=jnp.float32)
    sw = jnp.maximum(sw_lin, 0.0) + jnp.log1p(jnp.exp(-jnp.abs(sw_lin)))

    m_out[...] = m
    misc_out[...] = jnp.concatenate(
        [kcorr, smask_sum, pcnt, pmul, sw], axis=1)


def _tca_phase(spf, pool_i, sidx45, scw, cidx45, cw, postab, swt):
    bb = 128
    grid = (B // bb,)
    bspecs = [
        pl.BlockSpec((bb, SROWS), lambda i: (i, 0)),
        pl.BlockSpec((bb, PICKED), lambda i: (i, 0)),
        pl.BlockSpec((bb, SEEN * 4), lambda i: (i, 0)),
        pl.BlockSpec((bb, SEEN * 4), lambda i: (i, 0)),
        pl.BlockSpec((bb, 4), lambda i: (i, 0)),
        pl.BlockSpec((bb, 4), lambda i: (i, 0)),
        pl.BlockSpec((SEEN, SPD), lambda i: (0, 0)),
        pl.BlockSpec((SEEN, 3), lambda i: (0, 0)),
    ]
    return pl.pallas_call(
        _tca_body,
        grid=grid,
        in_specs=bspecs,
        out_specs=[pl.BlockSpec((bb, SEEN), lambda i: (i, 0)),
                   pl.BlockSpec((bb, 7), lambda i: (i, 0)),
                   pl.BlockSpec((bb, SPD), lambda i: (i, 0))],
        out_shape=[jax.ShapeDtypeStruct((B, SEEN), jnp.float32),
                   jax.ShapeDtypeStruct((B, 7), jnp.float32),
                   jax.ShapeDtypeStruct((B, SPD), jnp.float32)],
    )(spf, pool_i, sidx45, scw, cidx45, cw, postab, swt)


def _tcb_body(cce_ref, psum_ref, praw_ref, e0_ref, m_ref, misc_ref, pos_ref,
              cc_ref, wpack_ref, bpack_ref, wcst_ref, wrate_ref, brate_ref,
              out_ref):
    e0 = e0_ref[...]
    misc = misc_ref[...]
    kcorr, smask_sum = misc[:, 0:1], misc[:, 1:2]
    pcnt, pmul = misc[:, 2:3], misc[:, 3:4]
    sw0, sw1, sw2 = misc[:, 4:5], misc[:, 5:6], misc[:, 6:7]

    pool_ctx = (praw_ref[...] - (float(PICKED) - pcnt) * e0) * pmul

    m = m_ref[...]
    seen_wsum = jnp.sum(psum_ref[...] * m[:, :, None], axis=1) - kcorr * e0

    a = jnp.dot(seen_wsum, wpack_ref[...],
                preferred_element_type=jnp.float32)
    seen_ctx = (a + bpack_ref[...] * smask_sum + pos_ref[...]) / (
        smask_sum + 1e-9)
    v = jnp.dot(seen_ctx, wcst_ref[...],
                preferred_element_type=jnp.float32)

    u = (sw0 * 0.125) * pool_ctx + (sw1 * RSQ_SPD) * v \
        + sw2 * wrate_ref[...]

    cce = cce_ref[...]
    scores = jnp.sum(cce * u[:, None, :], axis=2) + sw2 * brate_ref[0, 0]
    mask = (cc_ref[...] > 0).astype(jnp.float32)
    out_ref[...] = scores * mask


def _tcb_phase(cce, psums, praw, e0row, m, misc, pos, cc,
               wpack, bpack, wcst, wrate, brate):
    nb = cce.shape[0]
    bb = 128
    grid = (nb // bb,)
    bspecs = [
        pl.BlockSpec((bb, PACK, D), lambda i: (i, 0, 0)),
        pl.BlockSpec((bb, SEEN, D), lambda i: (i, 0, 0)),
        pl.BlockSpec((bb, D), lambda i: (i, 0)),
        pl.BlockSpec((1, D), lambda i: (0, 0)),
        pl.BlockSpec((bb, SEEN), lambda i: (i, 0)),
        pl.BlockSpec((bb, 7), lambda i: (i, 0)),
        pl.BlockSpec((bb, SPD), lambda i: (i, 0)),
        pl.BlockSpec((bb, PACK), lambda i: (i, 0)),
        pl.BlockSpec((D, SPD), lambda i: (0, 0)),
        pl.BlockSpec((1, SPD), lambda i: (0, 0)),
        pl.BlockSpec((SPD, D), lambda i: (0, 0)),
        pl.BlockSpec((1, D), lambda i: (0, 0)),
        pl.BlockSpec((1, 1), lambda i: (0, 0)),
    ]
    return pl.pallas_call(
        _tcb_body,
        grid=grid,
        in_specs=bspecs,
        out_specs=pl.BlockSpec((bb, PACK), lambda i: (i, 0)),
        out_shape=jax.ShapeDtypeStruct((nb, PACK), jnp.float32),
    )(cce, psums, praw, e0row, m, misc, pos, cc,
      wpack, bpack, wcst, wrate, brate)


def kernel(card_choices, pool, seen_packs, seen_coords, seen_coord_weights,
           coords, coord_weights, card_embeddings, W_pack, b_pack,
           pack_pos_table, W_card_seen, W_rate, b_rate,
           sublayer_weight_table):
    cc = card_choices.astype(jnp.int32)
    pool_i = pool.astype(jnp.int32)
    spf = seen_packs.astype(jnp.int32).reshape(B, SROWS)
    idx_all = jnp.concatenate([spf, pool_i, cc], axis=1)

    (sc1, sc2, e0row) = _sc_phase_split(card_embeddings, idx_all)

    sidx45 = (seen_coords[..., 0] * 15 + seen_coords[..., 1]) \
        .astype(jnp.int32).reshape(B, SEEN * 4)
    cidx45 = (coords[..., 0] * 15 + coords[..., 1]).astype(jnp.int32)
    scw = seen_coord_weights.reshape(B, SEEN * 4)

    m, misc, pos = _tca_phase(spf, pool_i, sidx45, scw, cidx45, coord_weights,
                              pack_pos_table, sublayer_weight_table)

    h = B // 2
    wargs = (W_pack, b_pack.reshape(1, SPD), W_card_seen.T,
             W_rate[:, 0].reshape(1, D), b_rate.reshape(1, 1))
    s1 = _tcb_phase(sc1[0], sc1[1], sc1[2], e0row, m[:h], misc[:h], pos[:h],
                    cc[:h], *wargs)
    s2 = _tcb_phase(sc2[0], sc2[1], sc2[2], e0row, m[h:], misc[h:], pos[h:],
                    cc[h:], *wargs)
    return jnp.concatenate([s1, s2], axis=0)
